# baseline (speedup 1.0000x reference)
import functools

import jax
import jax.numpy as jnp
from jax import lax
from jax.experimental import pallas as pl
from jax.experimental.pallas import tpu as pltpu
from jax.experimental.pallas import tpu_sc as plsc

_K = 16
_QT = 256
_C = 1024

_NEG = float("-inf")
_IMAX = jnp.iinfo(jnp.int32).max


def _extract_topk(vals, idx, nk):
    out_v, out_i = [], []
    for _ in range(nk):
        m = jnp.max(vals, axis=1, keepdims=True)
        am = jnp.min(jnp.where(vals == m, idx, _IMAX), axis=1, keepdims=True)
        out_v.append(m)
        out_i.append(am)
        vals = jnp.where(idx == am, _NEG, vals)
    return jnp.concatenate(out_v, axis=1), jnp.concatenate(out_i, axis=1)


def _topk_body(n_valid, x_ref, mem_ref, out_ref, vals_s, idx_s):
    j = pl.program_id(1)

    @pl.when(j == 0)
    def _init():
        vals_s[...] = jnp.full(vals_s.shape, _NEG, vals_s.dtype)
        idx_s[...] = jnp.full(idx_s.shape, _IMAX, idx_s.dtype)

    sims = lax.dot_general(x_ref[...], mem_ref[...],
                           (((1,), (1,)), ((), ())),
                           preferred_element_type=jnp.float32)
    gcol = j * _C + lax.broadcasted_iota(jnp.int32, sims.shape, 1)
    sims = jnp.where(gcol < n_valid, sims, _NEG)
    cv, ci = _extract_topk(sims, gcol, _K)
    mv = jnp.concatenate([vals_s[...], cv], axis=1)
    mi = jnp.concatenate([idx_s[...], ci], axis=1)
    nv, ni = _extract_topk(mv, mi, _K)
    vals_s[...] = nv
    idx_s[...] = ni

    @pl.when(j == pl.num_programs(1) - 1)
    def _emit():
        out_ref[...] = ni


def _topk_indices(x, mem_pad, n_valid):
    q, d = x.shape
    npad = mem_pad.shape[0]
    return pl.pallas_call(
        functools.partial(_topk_body, n_valid),
        grid=(q // _QT, npad // _C),
        in_specs=[pl.BlockSpec((_QT, d), lambda qi, j: (qi, 0)),
                  pl.BlockSpec((_C, d), lambda qi, j: (j, 0))],
        out_specs=pl.BlockSpec((_QT, _K), lambda qi, j: (qi, 0)),
        out_shape=jax.ShapeDtypeStruct((q, _K), jnp.int32),
        scratch_shapes=[pltpu.VMEM((_QT, _K), jnp.float32),
                        pltpu.VMEM((_QT, _K), jnp.int32)],
        compiler_params=pltpu.CompilerParams(
            dimension_semantics=("parallel", "arbitrary")),
    )(x, mem_pad)


def _gather_sum(mem_flat, eidx, d):
    info = plsc.get_sparse_core_info()
    nw = info.num_cores * info.num_subcores
    b = eidx.shape[0]
    epw = b // nw
    qpw = epw // (_K * d)
    mesh = plsc.VectorSubcoreMesh(core_axis_name="c", subcore_axis_name="s")

    @functools.partial(
        pl.kernel, mesh=mesh,
        out_type=jax.ShapeDtypeStruct((b // (_K * d), d), jnp.float32),
        scratch_types=[pltpu.VMEM((epw,), jnp.int32),
                       pltpu.VMEM((epw,), jnp.float32),
                       pltpu.VMEM((qpw, d), jnp.float32),
                       pltpu.SemaphoreType.DMA],
    )
    def gather_kernel(mem_hbm, eidx_hbm, out_hbm, eidx_v, vals_v, acc_v, sem):
        wid = lax.axis_index("s") * info.num_cores + lax.axis_index("c")
        pltpu.sync_copy(eidx_hbm.at[pl.ds(wid * epw, epw)], eidx_v)
        pltpu.async_copy(mem_hbm.at[eidx_v], vals_v, sem).wait()
        for qq in range(qpw):
            q0 = qq * _K * d
            acc = vals_v[pl.ds(q0, d)]
            for jj in range(1, _K):
                acc = acc + vals_v[pl.ds(q0 + jj * d, d)]
            acc_v[qq] = acc
        pltpu.sync_copy(acc_v, out_hbm.at[pl.ds(wid * qpw, qpw)])

    return gather_kernel(mem_flat, eidx)


def kernel(x, memory, k):
    n, d = memory.shape
    npad = ((n + _C - 1) // _C) * _C
    mem_pad = jnp.pad(memory, ((0, npad - n), (0, 0)))
    idx = _topk_indices(x, mem_pad, n)
    eidx = (idx[:, :, None] * d + jnp.arange(d, dtype=jnp.int32)).reshape(-1)
    sums = _gather_sum(memory.reshape(-1), eidx, d)
    return sums / k


# speedup vs baseline: 2.8565x; 2.8565x over previous
import functools

import jax
import jax.numpy as jnp
from jax import lax
from jax.experimental import pallas as pl
from jax.experimental.pallas import tpu as pltpu
from jax.experimental.pallas import tpu_sc as plsc

_K = 16
_QT = 256
_C = 4096
_NS = _C // 128

_NEG = float("-inf")
_IMAX = jnp.iinfo(jnp.int32).max


def _extract_topk(vals, idx, nk):
    out_v, out_i = [], []
    for _ in range(nk):
        m = jnp.max(vals, axis=1, keepdims=True)
        am = jnp.min(jnp.where(vals == m, idx, _IMAX), axis=1, keepdims=True)
        out_v.append(m)
        out_i.append(am)
        vals = jnp.where(idx == am, _NEG, vals)
    return jnp.concatenate(out_v, axis=1), jnp.concatenate(out_i, axis=1)


def _topk_body(n_valid, x_ref, mem_ref, out_ref, vals_s, idx_s):
    j = pl.program_id(1)

    @pl.when(j == 0)
    def _init():
        vals_s[...] = jnp.full(vals_s.shape, _NEG, vals_s.dtype)
        idx_s[...] = jnp.full(idx_s.shape, _IMAX, idx_s.dtype)

    sims = lax.dot_general(x_ref[...], mem_ref[...],
                           (((1,), (1,)), ((), ())),
                           preferred_element_type=jnp.float32)
    gcol = j * _C + lax.broadcasted_iota(jnp.int32, sims.shape, 1)
    sims = jnp.where(gcol < n_valid, sims, _NEG)

    bm = sims[:, 0:128]
    bidx = gcol[:, 0:128]
    for s in range(1, _NS):
        v = sims[:, s * 128:(s + 1) * 128]
        gi = gcol[:, s * 128:(s + 1) * 128]
        upd = v > bm
        bm = jnp.where(upd, v, bm)
        bidx = jnp.where(upd, gi, bidx)

    _, bi = _extract_topk(bm, bidx, _K)
    lane = bi & 127

    cvs = [vals_s[...]]
    cis = [idx_s[...]]
    for s in range(_NS):
        cvs.append(jnp.take_along_axis(sims[:, s * 128:(s + 1) * 128],
                                       lane, axis=1))
        cis.append(j * _C + s * 128 + lane)
    nv, ni = _extract_topk(jnp.concatenate(cvs, axis=1),
                           jnp.concatenate(cis, axis=1), _K)
    vals_s[...] = nv
    idx_s[...] = ni

    @pl.when(j == pl.num_programs(1) - 1)
    def _emit():
        out_ref[...] = ni


def _topk_indices(x, mem_pad, n_valid):
    q, d = x.shape
    npad = mem_pad.shape[0]
    return pl.pallas_call(
        functools.partial(_topk_body, n_valid),
        grid=(q // _QT, npad // _C),
        in_specs=[pl.BlockSpec((_QT, d), lambda qi, j: (qi, 0)),
                  pl.BlockSpec((_C, d), lambda qi, j: (j, 0))],
        out_specs=pl.BlockSpec((_QT, _K), lambda qi, j: (qi, 0)),
        out_shape=jax.ShapeDtypeStruct((q, _K), jnp.int32),
        scratch_shapes=[pltpu.VMEM((_QT, _K), jnp.float32),
                        pltpu.VMEM((_QT, _K), jnp.int32)],
        compiler_params=pltpu.CompilerParams(
            dimension_semantics=("parallel", "arbitrary")),
    )(x, mem_pad)


def _gather_sum(mem_flat, eidx, d):
    info = plsc.get_sparse_core_info()
    nw = info.num_cores * info.num_subcores
    b = eidx.shape[0]
    epw = b // nw
    qpw = epw // (_K * d)
    mesh = plsc.VectorSubcoreMesh(core_axis_name="c", subcore_axis_name="s")

    @functools.partial(
        pl.kernel, mesh=mesh,
        out_type=jax.ShapeDtypeStruct((b // (_K * d), d), jnp.float32),
        scratch_types=[pltpu.VMEM((epw,), jnp.int32),
                       pltpu.VMEM((epw,), jnp.float32),
                       pltpu.VMEM((qpw, d), jnp.float32),
                       pltpu.SemaphoreType.DMA],
    )
    def gather_kernel(mem_hbm, eidx_hbm, out_hbm, eidx_v, vals_v, acc_v, sem):
        wid = lax.axis_index("s") * info.num_cores + lax.axis_index("c")
        pltpu.sync_copy(eidx_hbm.at[pl.ds(wid * epw, epw)], eidx_v)
        pltpu.async_copy(mem_hbm.at[eidx_v], vals_v, sem).wait()
        for qq in range(qpw):
            q0 = qq * _K * d
            acc = vals_v[pl.ds(q0, d)]
            for jj in range(1, _K):
                acc = acc + vals_v[pl.ds(q0 + jj * d, d)]
            acc_v[qq] = acc
        pltpu.sync_copy(acc_v, out_hbm.at[pl.ds(wid * qpw, qpw)])

    return gather_kernel(mem_flat, eidx)


def kernel(x, memory, k):
    n, d = memory.shape
    npad = ((n + _C - 1) // _C) * _C
    mem_pad = jnp.pad(memory, ((0, npad - n), (0, 0)))
    idx = _topk_indices(x, mem_pad, n)
    eidx = (idx[:, :, None] * d + jnp.arange(d, dtype=jnp.int32)).reshape(-1)
    sums = _gather_sum(memory.reshape(-1), eidx, d)
    return sums / k


# speedup vs baseline: 4.5334x; 1.5871x over previous
import functools

import jax
import jax.numpy as jnp
from jax import lax
from jax.experimental import pallas as pl
from jax.experimental.pallas import tpu as pltpu
from jax.experimental.pallas import tpu_sc as plsc

_K = 16
_QT = 256
_C = 8192
_NS = _C // 128
_CW = _NS * _K

_NEG = float("-inf")
_IMAX = jnp.iinfo(jnp.int32).max


_FBIG = float(2.0**30)


def _extract_topk(vals, keyf, nk):
    out_v, out_i = [], []
    for _ in range(nk):
        m = jnp.max(vals, axis=1, keepdims=True)
        am = jnp.min(jnp.where(vals == m, keyf, _FBIG), axis=1, keepdims=True)
        out_v.append(m)
        out_i.append(am)
        vals = jnp.where(keyf == am, _NEG, vals)
    return jnp.concatenate(out_v, axis=1), jnp.concatenate(out_i, axis=1)


def _fold_best(vals, idx, nslices):
    bm = vals[:, 0:128]
    bi = idx[:, 0:128]
    for s in range(1, nslices):
        v = vals[:, s * 128:(s + 1) * 128]
        gi = idx[:, s * 128:(s + 1) * 128]
        upd = v > bm
        bm = jnp.where(upd, v, bm)
        bi = jnp.where(upd, gi, bi)
    return bm, bi


def _topk_body(n_valid, x_ref, mem_ref, out_ref, vals_s, idx_s):
    j = pl.program_id(1)

    @pl.when(j == 0)
    def _init():
        vals_s[...] = jnp.full(vals_s.shape, _NEG, vals_s.dtype)
        idx_s[...] = jnp.full(idx_s.shape, _IMAX, idx_s.dtype)

    sims = lax.dot_general(x_ref[...], mem_ref[...],
                           (((1,), (1,)), ((), ())),
                           preferred_element_type=jnp.float32)
    lcol = lax.broadcasted_iota(jnp.int32, sims.shape, 1)
    sims = jnp.where(j * _C + lcol < n_valid, sims, _NEG)

    lcolf = lcol.astype(jnp.float32)
    bm, bi = _fold_best(sims, lcolf, _NS)
    _, b1 = _extract_topk(bm, bi, _K)
    lane1 = b1.astype(jnp.int32) & 127
    cvs = []
    for s in range(_NS):
        sl = slice(s * 128, (s + 1) * 128)
        cvs.append(jnp.take_along_axis(sims[:, sl], lane1, axis=1))
    cand_v = jnp.concatenate(cvs, axis=1)
    ccol = lax.broadcasted_iota(jnp.int32, cand_v.shape, 1)
    cand_loc = ((ccol >> 4) << 7) + jnp.tile(lane1, (1, _NS))
    cand_i = j * _C + cand_loc

    fv = jnp.concatenate([vals_s[...], cand_v], axis=1)
    fkey = jnp.concatenate(
        [idx_s[...].astype(jnp.float32), cand_i.astype(jnp.float32)], axis=1)
    nv, nif = _extract_topk(fv, fkey, _K)
    ni = nif.astype(jnp.int32)
    vals_s[...] = nv
    idx_s[...] = ni

    @pl.when(j == pl.num_programs(1) - 1)
    def _emit():
        out_ref[...] = ni


def _topk_indices(x, mem_pad, n_valid):
    q, d = x.shape
    npad = mem_pad.shape[0]
    return pl.pallas_call(
        functools.partial(_topk_body, n_valid),
        grid=(q // _QT, npad // _C),
        in_specs=[pl.BlockSpec((_QT, d), lambda qi, j: (qi, 0)),
                  pl.BlockSpec((_C, d), lambda qi, j: (j, 0))],
        out_specs=pl.BlockSpec((_QT, _K), lambda qi, j: (qi, 0)),
        out_shape=jax.ShapeDtypeStruct((q, _K), jnp.int32),
        scratch_shapes=[pltpu.VMEM((_QT, _K), jnp.float32),
                        pltpu.VMEM((_QT, _K), jnp.int32)],
        compiler_params=pltpu.CompilerParams(
            dimension_semantics=("parallel", "arbitrary")),
    )(x, mem_pad)


def _gather_sum(mem_flat, eidx, d):
    info = plsc.get_sparse_core_info()
    nw = info.num_cores * info.num_subcores
    b = eidx.shape[0]
    epw = b // nw
    qpw = epw // (_K * d)
    mesh = plsc.VectorSubcoreMesh(core_axis_name="c", subcore_axis_name="s")

    @functools.partial(
        pl.kernel, mesh=mesh,
        out_type=jax.ShapeDtypeStruct((b // (_K * d), d), jnp.float32),
        scratch_types=[pltpu.VMEM((epw,), jnp.int32),
                       pltpu.VMEM((epw,), jnp.float32),
                       pltpu.VMEM((qpw, d), jnp.float32),
                       pltpu.SemaphoreType.DMA],
    )
    def gather_kernel(mem_hbm, eidx_hbm, out_hbm, eidx_v, vals_v, acc_v, sem):
        wid = lax.axis_index("s") * info.num_cores + lax.axis_index("c")
        pltpu.sync_copy(eidx_hbm.at[pl.ds(wid * epw, epw)], eidx_v)
        pltpu.async_copy(mem_hbm.at[eidx_v], vals_v, sem).wait()
        for qq in range(qpw):
            q0 = qq * _K * d
            acc = vals_v[pl.ds(q0, d)]
            for jj in range(1, _K):
                acc = acc + vals_v[pl.ds(q0 + jj * d, d)]
            acc_v[qq] = acc
        pltpu.sync_copy(acc_v, out_hbm.at[pl.ds(wid * qpw, qpw)])

    return gather_kernel(mem_flat, eidx)


def kernel(x, memory, k):
    n, d = memory.shape
    npad = ((n + _C - 1) // _C) * _C
    mem_pad = jnp.pad(memory, ((0, npad - n), (0, 0)))
    idx = _topk_indices(x, mem_pad, n)
    eidx = (idx[:, :, None] * d + jnp.arange(d, dtype=jnp.int32)).reshape(-1)
    sums = _gather_sum(memory.reshape(-1), eidx, d)
    return sums / k
